# baseline (speedup 1.0000x reference)
import functools

import jax
import jax.numpy as jnp
from jax import lax
from jax.experimental import pallas as pl
from jax.experimental.pallas import tpu as pltpu
from jax.experimental.pallas import tpu_sc as plsc

N = 10000
E = 320000
D = 128
G = 64

NC = 2
NS = 16
NW = NC * NS
EPW = E // NW
K = 128
CPW = 80
CPH = CPW // 2
E_PAD = NW * CPW * K
RPT = 624
NP = 15 * RPT + 664


def _segsum_body(h_hbm, se_hbm, out_hbm, idx, rowsb, acc, g0, g1, s0, s1):
    c = lax.axis_index("c")
    s = lax.axis_index("s")
    wid = s * NC + c

    rows0 = rowsb.at[0]
    rows1 = rowsb.at[1]

    zv = jnp.zeros((16,), jnp.float32)

    def zrow(r, carry):
        for j in range(D // 16):
            rowsb[0, r, pl.ds(j * 16, 16)] = zv
        return carry

    lax.fori_loop(0, K, zrow, 0)

    base = s * RPT

    @pl.when(s < NS - 1)
    def _():
        for off, size in ((0, 128), (128, 128), (256, 128), (384, 128), (512, 112)):
            pltpu.sync_copy(rowsb.at[0, pl.ds(0, size)], acc.at[pl.ds(base + off, size)])

    @pl.when(s == NS - 1)
    def _():
        for off, size in ((0, 128), (128, 128), (256, 128), (384, 128), (512, 128), (640, 24)):
            pltpu.sync_copy(rowsb.at[0, pl.ds(0, size)], acc.at[pl.ds(base + off, size)])

    for half in range(2):
        pltpu.sync_copy(se_hbm.at[:, pl.ds(wid * CPW + half * CPH, CPH)], idx)
        pltpu.async_copy(h_hbm.at[idx.at[0, 0]], rows0, g0)
        pltpu.async_copy(h_hbm.at[idx.at[0, 1]], rows1, g1)
        if half == 0:
            plsc.subcore_barrier()

        def body(jj, carry):
            j0 = 2 * jj
            j1 = j0 + 1
            pltpu.make_async_copy(h_hbm.at[idx.at[0, j0]], rows0, g0).wait()
            pltpu.async_copy(rows0, acc.at[idx.at[1, j0]], s0, add=True)
            pltpu.make_async_copy(h_hbm.at[idx.at[0, j1]], rows1, g1).wait()
            pltpu.async_copy(rows1, acc.at[idx.at[1, j1]], s1, add=True)
            pltpu.make_async_copy(rows0, acc.at[idx.at[1, j0]], s0).wait()
            pltpu.async_copy(h_hbm.at[idx.at[0, j0 + 2]], rows0, g0)
            pltpu.make_async_copy(rows1, acc.at[idx.at[1, j1]], s1).wait()
            pltpu.async_copy(h_hbm.at[idx.at[0, j1 + 2]], rows1, g1)
            return carry

        lax.fori_loop(0, CPH // 2 - 1, body, 0)

        jt = CPH - 2
        pltpu.make_async_copy(h_hbm.at[idx.at[0, jt]], rows0, g0).wait()
        pltpu.async_copy(rows0, acc.at[idx.at[1, jt]], s0, add=True)
        pltpu.make_async_copy(h_hbm.at[idx.at[0, jt + 1]], rows1, g1).wait()
        pltpu.async_copy(rows1, acc.at[idx.at[1, jt + 1]], s1, add=True)
        pltpu.make_async_copy(rows0, acc.at[idx.at[1, jt]], s0).wait()
        pltpu.make_async_copy(rows1, acc.at[idx.at[1, jt + 1]], s1).wait()

    plsc.subcore_barrier()

    @pl.when(s < NS - 1)
    def _():
        pltpu.sync_copy(acc.at[pl.ds(base, RPT)], out_hbm.at[c, pl.ds(base, RPT)])

    @pl.when(s == NS - 1)
    def _():
        pltpu.sync_copy(acc.at[pl.ds(base, 664)], out_hbm.at[c, pl.ds(base, 664)])


@functools.cache
def _make_segsum():
    return pl.kernel(
        _segsum_body,
        out_type=jax.ShapeDtypeStruct((NC, NP, D), jnp.float32),
        mesh=plsc.VectorSubcoreMesh(
            core_axis_name="c", subcore_axis_name="s", num_cores=NC, num_subcores=NS
        ),
        scratch_types=[
            pltpu.VMEM((2, CPH, K), jnp.int32),
            pltpu.VMEM((2, K, D), jnp.float32),
            pltpu.VMEM_SHARED((NP, D), jnp.float32),
            pltpu.SemaphoreType.DMA,
            pltpu.SemaphoreType.DMA,
            pltpu.SemaphoreType.DMA,
            pltpu.SemaphoreType.DMA,
        ],
    )


def _segsum(h, se):
    return _make_segsum()(h, se)


R = 2000


def _mlp_body(h_ref, p_ref, w1_ref, b1_ref, w2_ref, b2_ref, m_ref, st_ref):
    z = h_ref[...] + p_ref[0] + p_ref[1]
    a = jnp.maximum(
        jnp.dot(z, w1_ref[...], preferred_element_type=jnp.float32) + b1_ref[...],
        0.0,
    )
    m = jnp.dot(a, w2_ref[...], preferred_element_type=jnp.float32) + b2_ref[...]
    m_ref[...] = m

    ssum = jnp.sum(m, axis=0, keepdims=True)
    ssq = jnp.sum(m * m, axis=0, keepdims=True)
    st = jnp.concatenate([ssum, ssq, jnp.zeros((6, D), jnp.float32)], axis=0)

    @pl.when(pl.program_id(0) == 0)
    def _():
        st_ref[...] = jnp.zeros_like(st_ref)

    st_ref[...] += st


def _mlp(h, p, w1, b1, w2, b2):
    return pl.pallas_call(
        _mlp_body,
        grid=(N // R,),
        in_specs=[
            pl.BlockSpec((R, D), lambda i: (i, 0)),
            pl.BlockSpec((NC, R, D), lambda i: (0, i, 0)),
            pl.BlockSpec((D, D), lambda i: (0, 0)),
            pl.BlockSpec((1, D), lambda i: (0, 0)),
            pl.BlockSpec((D, D), lambda i: (0, 0)),
            pl.BlockSpec((1, D), lambda i: (0, 0)),
        ],
        out_specs=[
            pl.BlockSpec((R, D), lambda i: (i, 0)),
            pl.BlockSpec((8, D), lambda i: (0, 0)),
        ],
        out_shape=[
            jax.ShapeDtypeStruct((N, D), jnp.float32),
            jax.ShapeDtypeStruct((8, D), jnp.float32),
        ],
    )(h, p, w1, b1, w2, b2)


def _bn_body(m_ref, st_ref, g_ref, b_ref, o_ref):
    mu = st_ref[0:1, :] * (1.0 / N)
    ex2 = st_ref[1:2, :] * (1.0 / N)
    var = ex2 - mu * mu
    v = (m_ref[...] - mu) * lax.rsqrt(var + 1e-5) * g_ref[...] + b_ref[...]
    o_ref[...] = jnp.maximum(v, 0.0)


def _bn_relu(m, st, g, b):
    return pl.pallas_call(
        _bn_body,
        grid=(N // R,),
        in_specs=[
            pl.BlockSpec((R, D), lambda i: (i, 0)),
            pl.BlockSpec((8, D), lambda i: (0, 0)),
            pl.BlockSpec((1, D), lambda i: (0, 0)),
            pl.BlockSpec((1, D), lambda i: (0, 0)),
        ],
        out_specs=pl.BlockSpec((R, D), lambda i: (i, 0)),
        out_shape=jax.ShapeDtypeStruct((N, D), jnp.float32),
    )(m, st, g, b)


def _soft_body(m_ref, st_ref, g_ref, b_ref, batch_ref, o_ref):
    mu = st_ref[0:1, :] * (1.0 / N)
    ex2 = st_ref[1:2, :] * (1.0 / N)
    var = ex2 - mu * mu
    v = (m_ref[...] - mu) * lax.rsqrt(var + 1e-5) * g_ref[...] + b_ref[...]
    s0 = jnp.broadcast_to(v[:, 0:1] * (1.0 / 5.0), (N, D))
    gid = lax.broadcasted_iota(jnp.int32, (N, D), 1)
    onehot = batch_ref[...] == gid
    neg = jnp.full((N, D), -jnp.inf, jnp.float32)
    mx_g = jnp.max(jnp.where(onehot, s0, neg), axis=0, keepdims=True)
    mxb = jnp.sum(
        jnp.where(onehot, jnp.broadcast_to(mx_g, (N, D)), 0.0), axis=1, keepdims=True
    )
    e = jnp.exp(s0 - jnp.broadcast_to(mxb, (N, D)))
    den_g = jnp.sum(jnp.where(onehot, e, 0.0), axis=0, keepdims=True)
    denb = jnp.sum(
        jnp.where(onehot, jnp.broadcast_to(den_g, (N, D)), 0.0), axis=1, keepdims=True
    )
    o_ref[...] = e / (jnp.broadcast_to(denb, (N, D)) + 1e-16)


def _softmax(m, st, g, b, batch2d):
    return pl.pallas_call(
        _soft_body,
        out_shape=jax.ShapeDtypeStruct((N, D), jnp.float32),
    )(m, st, g, b, batch2d)


def kernel(x, edge_index, batch,
           l0_w1, l0_b1, l0_w2, l0_b2,
           l1_w1, l1_b1, l1_w2, l1_b2,
           l2_w1, l2_b1, l2_w2, l2_b2,
           bn0_g, bn0_b, bn1_g, bn1_b, bn2_g, bn2_b):
    pad = E_PAD - E
    pad_src = (jnp.arange(pad, dtype=jnp.int32) * 13) % N
    pad_dst = N + jnp.arange(pad, dtype=jnp.int32) % (NP - N)
    se = jnp.stack([
        jnp.concatenate([edge_index[0], pad_src]),
        jnp.concatenate([edge_index[1], pad_dst]),
    ]).reshape(2, E_PAD // K, K)

    def row(a):
        return a.reshape(1, -1)

    p = _segsum(x, se)
    m, st = _mlp(x, p, l0_w1, row(l0_b1), l0_w2, row(l0_b2))
    h = _bn_relu(m, st, row(bn0_g), row(bn0_b))
    p = _segsum(h, se)
    m, st = _mlp(h, p, l1_w1, row(l1_b1), l1_w2, row(l1_b2))
    h = _bn_relu(m, st, row(bn1_g), row(bn1_b))
    p = _segsum(h, se)
    w2p = jnp.pad(l2_w2, ((0, 0), (0, D - 1)))
    b2p = jnp.pad(l2_b2.reshape(1, 1), ((0, 0), (0, D - 1)))
    m, st = _mlp(h, p, l2_w1, row(l2_b1), w2p, b2p)
    g2 = jnp.broadcast_to(bn2_g.reshape(1, 1), (1, D))
    b2 = jnp.broadcast_to(bn2_b.reshape(1, 1), (1, D))
    out = _softmax(m, st, g2, b2, batch.reshape(N, 1))
    return out[:, :1]


# speedup vs baseline: 1.0067x; 1.0067x over previous
import functools

import jax
import jax.numpy as jnp
from jax import lax
from jax.experimental import pallas as pl
from jax.experimental.pallas import tpu as pltpu
from jax.experimental.pallas import tpu_sc as plsc

N = 10000
E = 320000
D = 128
G = 64

NC = 2
NS = 16
NW = NC * NS
EPW = E // NW
K = 128
CPW = 80
CPH = CPW // 2
E_PAD = NW * CPW * K
RPT = 624
NP = 15 * RPT + 664


def _segsum_body(h_hbm, se_hbm, out_hbm, idx, rowsb, acc, g0, g1, s0, s1):
    c = lax.axis_index("c")
    s = lax.axis_index("s")
    wid = s * NC + c

    rows0 = rowsb.at[0]
    rows1 = rowsb.at[1]

    zv = jnp.zeros((16,), jnp.float32)

    def zrow(r, carry):
        for j in range(D // 16):
            rowsb[0, r, pl.ds(j * 16, 16)] = zv
        return carry

    lax.fori_loop(0, K, zrow, 0)

    base = s * RPT

    @pl.when(s < NS - 1)
    def _():
        for off, size in ((0, 128), (128, 128), (256, 128), (384, 128), (512, 112)):
            pltpu.sync_copy(rowsb.at[0, pl.ds(0, size)], acc.at[pl.ds(base + off, size)])

    @pl.when(s == NS - 1)
    def _():
        for off, size in ((0, 128), (128, 128), (256, 128), (384, 128), (512, 128), (640, 24)):
            pltpu.sync_copy(rowsb.at[0, pl.ds(0, size)], acc.at[pl.ds(base + off, size)])

    for half in range(2):
        pltpu.sync_copy(se_hbm.at[:, pl.ds(wid * CPW + half * CPH, CPH)], idx)
        pltpu.async_copy(h_hbm.at[idx.at[0, 0]], rows0, g0)
        pltpu.async_copy(h_hbm.at[idx.at[0, 1]], rows1, g1)
        if half == 0:
            plsc.subcore_barrier()

        def body(jj, carry):
            j0 = 2 * jj
            j1 = j0 + 1
            pltpu.make_async_copy(h_hbm.at[idx.at[0, j0]], rows0, g0).wait()
            pltpu.async_copy(rows0, acc.at[idx.at[1, j0]], s0, add=True)
            pltpu.make_async_copy(h_hbm.at[idx.at[0, j1]], rows1, g1).wait()
            pltpu.async_copy(rows1, acc.at[idx.at[1, j1]], s1, add=True)
            pltpu.make_async_copy(rows0, acc.at[idx.at[1, j0]], s0).wait()
            pltpu.async_copy(h_hbm.at[idx.at[0, j0 + 2]], rows0, g0)
            pltpu.make_async_copy(rows1, acc.at[idx.at[1, j1]], s1).wait()
            pltpu.async_copy(h_hbm.at[idx.at[0, j1 + 2]], rows1, g1)
            return carry

        lax.fori_loop(0, CPH // 2 - 1, body, 0)

        jt = CPH - 2
        pltpu.make_async_copy(h_hbm.at[idx.at[0, jt]], rows0, g0).wait()
        pltpu.async_copy(rows0, acc.at[idx.at[1, jt]], s0, add=True)
        pltpu.make_async_copy(h_hbm.at[idx.at[0, jt + 1]], rows1, g1).wait()
        pltpu.async_copy(rows1, acc.at[idx.at[1, jt + 1]], s1, add=True)
        pltpu.make_async_copy(rows0, acc.at[idx.at[1, jt]], s0).wait()
        pltpu.make_async_copy(rows1, acc.at[idx.at[1, jt + 1]], s1).wait()

    plsc.subcore_barrier()

    @pl.when(s < NS - 1)
    def _():
        pltpu.sync_copy(acc.at[pl.ds(base, RPT)], out_hbm.at[c, pl.ds(base, RPT)])

    @pl.when(s == NS - 1)
    def _():
        pltpu.sync_copy(acc.at[pl.ds(base, 664)], out_hbm.at[c, pl.ds(base, 664)])


@functools.cache
def _make_segsum():
    return pl.kernel(
        _segsum_body,
        out_type=jax.ShapeDtypeStruct((NC, NP, D), jnp.float32),
        mesh=plsc.VectorSubcoreMesh(
            core_axis_name="c", subcore_axis_name="s", num_cores=NC, num_subcores=NS
        ),
        scratch_types=[
            pltpu.VMEM((2, CPH, K), jnp.int32),
            pltpu.VMEM((2, K, D), jnp.float32),
            pltpu.VMEM_SHARED((NP, D), jnp.float32),
            pltpu.SemaphoreType.DMA,
            pltpu.SemaphoreType.DMA,
            pltpu.SemaphoreType.DMA,
            pltpu.SemaphoreType.DMA,
        ],
    )


def _segsum(h, se):
    return _make_segsum()(h, se)


R = 2000
NB = N // R


def _mlp_bn_body(h_ref, p_ref, w1_ref, b1_ref, w2_ref, b2_ref, g_ref, b_ref,
                 o_ref, m_s, st_s):
    ph = pl.program_id(0)
    i = pl.program_id(1)

    @pl.when(ph == 0)
    def _():
        z = h_ref[...] + p_ref[0] + p_ref[1]
        a = jnp.maximum(
            jnp.dot(z, w1_ref[...], preferred_element_type=jnp.float32) + b1_ref[...],
            0.0,
        )
        m = jnp.dot(a, w2_ref[...], preferred_element_type=jnp.float32) + b2_ref[...]
        m_s[pl.ds(i * R, R), :] = m
        st = jnp.concatenate(
            [jnp.sum(m, axis=0, keepdims=True), jnp.sum(m * m, axis=0, keepdims=True)],
            axis=0,
        )

        @pl.when(i == 0)
        def _():
            st_s[...] = jnp.zeros_like(st_s)

        st_s[...] += st
        o_ref[...] = m

    @pl.when(ph == 1)
    def _():
        mu = st_s[0:1, :] * (1.0 / N)
        ex2 = st_s[1:2, :] * (1.0 / N)
        var = ex2 - mu * mu
        m = m_s[pl.ds(i * R, R), :]
        v = (m - mu) * lax.rsqrt(var + 1e-5) * g_ref[...] + b_ref[...]
        o_ref[...] = jnp.maximum(v, 0.0)


def _mlp_bn(h, p, w1, b1, w2, b2, g, b):
    return pl.pallas_call(
        _mlp_bn_body,
        grid=(2, NB),
        in_specs=[
            pl.BlockSpec((R, D), lambda ph, i: (i, 0)),
            pl.BlockSpec((NC, R, D), lambda ph, i: (0, i, 0)),
            pl.BlockSpec((D, D), lambda ph, i: (0, 0)),
            pl.BlockSpec((1, D), lambda ph, i: (0, 0)),
            pl.BlockSpec((D, D), lambda ph, i: (0, 0)),
            pl.BlockSpec((1, D), lambda ph, i: (0, 0)),
            pl.BlockSpec((1, D), lambda ph, i: (0, 0)),
            pl.BlockSpec((1, D), lambda ph, i: (0, 0)),
        ],
        out_specs=pl.BlockSpec((R, D), lambda ph, i: (i, 0)),
        out_shape=jax.ShapeDtypeStruct((N, D), jnp.float32),
        scratch_shapes=[
            pltpu.VMEM((N, D), jnp.float32),
            pltpu.VMEM((2, D), jnp.float32),
        ],
    )(h, p, w1, b1, w2, b2, g, b)


def _mlp_soft_body(h_ref, p_ref, w1_ref, b1_ref, w2_ref, b2_ref, g_ref, b_ref,
                   batch_ref, o_ref, m_s, st_s):
    ph = pl.program_id(0)
    i = pl.program_id(1)

    @pl.when(ph == 0)
    def _():
        z = h_ref[...] + p_ref[0] + p_ref[1]
        a = jnp.maximum(
            jnp.dot(z, w1_ref[...], preferred_element_type=jnp.float32) + b1_ref[...],
            0.0,
        )
        m = jnp.dot(a, w2_ref[...], preferred_element_type=jnp.float32) + b2_ref[...]
        m_s[pl.ds(i * R, R), :] = m
        st = jnp.concatenate(
            [jnp.sum(m, axis=0, keepdims=True), jnp.sum(m * m, axis=0, keepdims=True)],
            axis=0,
        )

        @pl.when(i == 0)
        def _():
            st_s[...] = jnp.zeros_like(st_s)

        st_s[...] += st

    @pl.when((ph == 1) & (i == 0))
    def _():
        mu = st_s[0:1, :] * (1.0 / N)
        ex2 = st_s[1:2, :] * (1.0 / N)
        var = ex2 - mu * mu
        v = (m_s[...] - mu) * lax.rsqrt(var + 1e-5) * g_ref[...] + b_ref[...]
        s0 = jnp.broadcast_to(v[:, 0:1] * (1.0 / 5.0), (N, D))
        gid = lax.broadcasted_iota(jnp.int32, (N, D), 1)
        onehot = batch_ref[...] == gid
        neg = jnp.full((N, D), -jnp.inf, jnp.float32)
        mx_g = jnp.max(jnp.where(onehot, s0, neg), axis=0, keepdims=True)
        mxb = jnp.sum(
            jnp.where(onehot, jnp.broadcast_to(mx_g, (N, D)), 0.0),
            axis=1, keepdims=True,
        )
        e = jnp.exp(s0 - jnp.broadcast_to(mxb, (N, D)))
        den_g = jnp.sum(jnp.where(onehot, e, 0.0), axis=0, keepdims=True)
        denb = jnp.sum(
            jnp.where(onehot, jnp.broadcast_to(den_g, (N, D)), 0.0),
            axis=1, keepdims=True,
        )
        o_ref[...] = e / (jnp.broadcast_to(denb, (N, D)) + 1e-16)


def _mlp_soft(h, p, w1, b1, w2, b2, g, b, batch2d):
    return pl.pallas_call(
        _mlp_soft_body,
        grid=(2, NB),
        in_specs=[
            pl.BlockSpec((R, D), lambda ph, i: (i, 0)),
            pl.BlockSpec((NC, R, D), lambda ph, i: (0, i, 0)),
            pl.BlockSpec((D, D), lambda ph, i: (0, 0)),
            pl.BlockSpec((1, D), lambda ph, i: (0, 0)),
            pl.BlockSpec((D, D), lambda ph, i: (0, 0)),
            pl.BlockSpec((1, D), lambda ph, i: (0, 0)),
            pl.BlockSpec((1, D), lambda ph, i: (0, 0)),
            pl.BlockSpec((1, D), lambda ph, i: (0, 0)),
            pl.BlockSpec((N, 1), lambda ph, i: (0, 0)),
        ],
        out_specs=pl.BlockSpec((N, D), lambda ph, i: (0, 0)),
        out_shape=jax.ShapeDtypeStruct((N, D), jnp.float32),
        scratch_shapes=[
            pltpu.VMEM((N, D), jnp.float32),
            pltpu.VMEM((2, D), jnp.float32),
        ],
    )(h, p, w1, b1, w2, b2, g, b, batch2d)


def kernel(x, edge_index, batch,
           l0_w1, l0_b1, l0_w2, l0_b2,
           l1_w1, l1_b1, l1_w2, l1_b2,
           l2_w1, l2_b1, l2_w2, l2_b2,
           bn0_g, bn0_b, bn1_g, bn1_b, bn2_g, bn2_b):
    pad = E_PAD - E
    pad_src = (jnp.arange(pad, dtype=jnp.int32) * 13) % N
    pad_dst = N + jnp.arange(pad, dtype=jnp.int32) % (NP - N)
    se = jnp.stack([
        jnp.concatenate([edge_index[0], pad_src]),
        jnp.concatenate([edge_index[1], pad_dst]),
    ]).reshape(2, E_PAD // K, K)

    def row(a):
        return a.reshape(1, -1)

    p = _segsum(x, se)
    h = _mlp_bn(x, p, l0_w1, row(l0_b1), l0_w2, row(l0_b2), row(bn0_g), row(bn0_b))
    p = _segsum(h, se)
    h = _mlp_bn(h, p, l1_w1, row(l1_b1), l1_w2, row(l1_b2), row(bn1_g), row(bn1_b))
    p = _segsum(h, se)
    w2p = jnp.pad(l2_w2, ((0, 0), (0, D - 1)))
    b2p = jnp.pad(l2_b2.reshape(1, 1), ((0, 0), (0, D - 1)))
    g2 = jnp.broadcast_to(bn2_g.reshape(1, 1), (1, D))
    b2 = jnp.broadcast_to(bn2_b.reshape(1, 1), (1, D))
    out = _mlp_soft(h, p, l2_w1, row(l2_b1), w2p, b2p, g2, b2, batch.reshape(N, 1))
    return out[:, :1]


# speedup vs baseline: 1.2715x; 1.2630x over previous
import functools

import jax
import jax.numpy as jnp
from jax import lax
from jax.experimental import pallas as pl
from jax.experimental.pallas import tpu as pltpu
from jax.experimental.pallas import tpu_sc as plsc

N = 10000
E = 320000
D = 128
G = 64

NC = 2
NS = 16
NW = NC * NS
EPW = E // NW
K = 128
CPW = 80
CPH = CPW // 2
E_PAD = NW * CPW * K
RPT = 624
NP = 15 * RPT + 664


def _segsum_body(h_hbm, se_hbm, out_hbm, idx, rowsb, acc, g0, g1):
    c = lax.axis_index("c")
    s = lax.axis_index("s")
    wid = s * NC + c

    rows0 = rowsb.at[0]
    rows1 = rowsb.at[1]

    zv = jnp.zeros((16,), jnp.float32)

    def zrow(r, carry):
        for j in range(D // 16):
            rowsb[0, r, pl.ds(j * 16, 16)] = zv
        return carry

    lax.fori_loop(0, K, zrow, 0)

    base = s * RPT

    @pl.when(s < NS - 1)
    def _():
        for off, size in ((0, 128), (128, 128), (256, 128), (384, 128), (512, 112)):
            pltpu.sync_copy(rowsb.at[0, pl.ds(0, size)], acc.at[pl.ds(base + off, size)])

    @pl.when(s == NS - 1)
    def _():
        for off, size in ((0, 128), (128, 128), (256, 128), (384, 128), (512, 128), (640, 24)):
            pltpu.sync_copy(rowsb.at[0, pl.ds(0, size)], acc.at[pl.ds(base + off, size)])

    for half in range(2):
        pltpu.sync_copy(se_hbm.at[:, pl.ds(wid * CPW + half * CPH, CPH)], idx)
        pltpu.async_copy(h_hbm.at[idx.at[0, 0]], rows0, g0)
        if half == 0:
            plsc.subcore_barrier()

        def body(jj, carry):
            j0 = 2 * jj
            j1 = j0 + 1
            pltpu.async_copy(h_hbm.at[idx.at[0, j1]], rows1, g1)
            pltpu.make_async_copy(h_hbm.at[idx.at[0, j0]], rows0, g0).wait()
            pltpu.sync_copy(rows0, acc.at[idx.at[1, j0]], add=True)

            @pl.when(j0 + 2 < CPH)
            def _():
                pltpu.async_copy(h_hbm.at[idx.at[0, j0 + 2]], rows0, g0)

            pltpu.make_async_copy(h_hbm.at[idx.at[0, j1]], rows1, g1).wait()
            pltpu.sync_copy(rows1, acc.at[idx.at[1, j1]], add=True)
            return carry

        lax.fori_loop(0, CPH // 2, body, 0)

    plsc.subcore_barrier()

    @pl.when(s < NS - 1)
    def _():
        pltpu.sync_copy(acc.at[pl.ds(base, RPT)], out_hbm.at[c, pl.ds(base, RPT)])

    @pl.when(s == NS - 1)
    def _():
        pltpu.sync_copy(acc.at[pl.ds(base, 664)], out_hbm.at[c, pl.ds(base, 664)])


@functools.cache
def _make_segsum():
    return pl.kernel(
        _segsum_body,
        out_type=jax.ShapeDtypeStruct((NC, NP, D), jnp.float32),
        mesh=plsc.VectorSubcoreMesh(
            core_axis_name="c", subcore_axis_name="s", num_cores=NC, num_subcores=NS
        ),
        scratch_types=[
            pltpu.VMEM((2, CPH, K), jnp.int32),
            pltpu.VMEM((2, K, D), jnp.float32),
            pltpu.VMEM_SHARED((NP, D), jnp.float32),
            pltpu.SemaphoreType.DMA,
            pltpu.SemaphoreType.DMA,
        ],
    )


def _segsum(h, se):
    return _make_segsum()(h, se)


R = 2000
NB = N // R


def _mlp_bn_body(h_ref, p_ref, w1_ref, b1_ref, w2_ref, b2_ref, g_ref, b_ref,
                 o_ref, m_s, st_s):
    ph = pl.program_id(0)
    i = pl.program_id(1)

    @pl.when(ph == 0)
    def _():
        z = h_ref[...] + p_ref[0] + p_ref[1]
        a = jnp.maximum(
            jnp.dot(z, w1_ref[...], preferred_element_type=jnp.float32) + b1_ref[...],
            0.0,
        )
        m = jnp.dot(a, w2_ref[...], preferred_element_type=jnp.float32) + b2_ref[...]
        m_s[pl.ds(i * R, R), :] = m
        st = jnp.concatenate(
            [jnp.sum(m, axis=0, keepdims=True), jnp.sum(m * m, axis=0, keepdims=True)],
            axis=0,
        )

        @pl.when(i == 0)
        def _():
            st_s[...] = jnp.zeros_like(st_s)

        st_s[...] += st
        o_ref[...] = m

    @pl.when(ph == 1)
    def _():
        mu = st_s[0:1, :] * (1.0 / N)
        ex2 = st_s[1:2, :] * (1.0 / N)
        var = ex2 - mu * mu
        m = m_s[pl.ds(i * R, R), :]
        v = (m - mu) * lax.rsqrt(var + 1e-5) * g_ref[...] + b_ref[...]
        o_ref[...] = jnp.maximum(v, 0.0)


def _mlp_bn(h, p, w1, b1, w2, b2, g, b):
    return pl.pallas_call(
        _mlp_bn_body,
        grid=(2, NB),
        in_specs=[
            pl.BlockSpec((R, D), lambda ph, i: (i, 0)),
            pl.BlockSpec((NC, R, D), lambda ph, i: (0, i, 0)),
            pl.BlockSpec((D, D), lambda ph, i: (0, 0)),
            pl.BlockSpec((1, D), lambda ph, i: (0, 0)),
            pl.BlockSpec((D, D), lambda ph, i: (0, 0)),
            pl.BlockSpec((1, D), lambda ph, i: (0, 0)),
            pl.BlockSpec((1, D), lambda ph, i: (0, 0)),
            pl.BlockSpec((1, D), lambda ph, i: (0, 0)),
        ],
        out_specs=pl.BlockSpec((R, D), lambda ph, i: (i, 0)),
        out_shape=jax.ShapeDtypeStruct((N, D), jnp.float32),
        scratch_shapes=[
            pltpu.VMEM((N, D), jnp.float32),
            pltpu.VMEM((2, D), jnp.float32),
        ],
    )(h, p, w1, b1, w2, b2, g, b)


def _mlp_soft_body(h_ref, p_ref, w1_ref, b1_ref, w2_ref, b2_ref, g_ref, b_ref,
                   batch_ref, o_ref, m_s, st_s):
    ph = pl.program_id(0)
    i = pl.program_id(1)

    @pl.when(ph == 0)
    def _():
        z = h_ref[...] + p_ref[0] + p_ref[1]
        a = jnp.maximum(
            jnp.dot(z, w1_ref[...], preferred_element_type=jnp.float32) + b1_ref[...],
            0.0,
        )
        m = jnp.dot(a, w2_ref[...], preferred_element_type=jnp.float32) + b2_ref[...]
        m_s[pl.ds(i * R, R), :] = m
        st = jnp.concatenate(
            [jnp.sum(m, axis=0, keepdims=True), jnp.sum(m * m, axis=0, keepdims=True)],
            axis=0,
        )

        @pl.when(i == 0)
        def _():
            st_s[...] = jnp.zeros_like(st_s)

        st_s[...] += st

    @pl.when((ph == 1) & (i == 0))
    def _():
        mu = st_s[0:1, :] * (1.0 / N)
        ex2 = st_s[1:2, :] * (1.0 / N)
        var = ex2 - mu * mu
        v = (m_s[...] - mu) * lax.rsqrt(var + 1e-5) * g_ref[...] + b_ref[...]
        s0 = jnp.broadcast_to(v[:, 0:1] * (1.0 / 5.0), (N, D))
        gid = lax.broadcasted_iota(jnp.int32, (N, D), 1)
        onehot = batch_ref[...] == gid
        neg = jnp.full((N, D), -jnp.inf, jnp.float32)
        mx_g = jnp.max(jnp.where(onehot, s0, neg), axis=0, keepdims=True)
        mxb = jnp.sum(
            jnp.where(onehot, jnp.broadcast_to(mx_g, (N, D)), 0.0),
            axis=1, keepdims=True,
        )
        e = jnp.exp(s0 - jnp.broadcast_to(mxb, (N, D)))
        den_g = jnp.sum(jnp.where(onehot, e, 0.0), axis=0, keepdims=True)
        denb = jnp.sum(
            jnp.where(onehot, jnp.broadcast_to(den_g, (N, D)), 0.0),
            axis=1, keepdims=True,
        )
        o_ref[...] = e / (jnp.broadcast_to(denb, (N, D)) + 1e-16)


def _mlp_soft(h, p, w1, b1, w2, b2, g, b, batch2d):
    return pl.pallas_call(
        _mlp_soft_body,
        grid=(2, NB),
        in_specs=[
            pl.BlockSpec((R, D), lambda ph, i: (i, 0)),
            pl.BlockSpec((NC, R, D), lambda ph, i: (0, i, 0)),
            pl.BlockSpec((D, D), lambda ph, i: (0, 0)),
            pl.BlockSpec((1, D), lambda ph, i: (0, 0)),
            pl.BlockSpec((D, D), lambda ph, i: (0, 0)),
            pl.BlockSpec((1, D), lambda ph, i: (0, 0)),
            pl.BlockSpec((1, D), lambda ph, i: (0, 0)),
            pl.BlockSpec((1, D), lambda ph, i: (0, 0)),
            pl.BlockSpec((N, 1), lambda ph, i: (0, 0)),
        ],
        out_specs=pl.BlockSpec((N, D), lambda ph, i: (0, 0)),
        out_shape=jax.ShapeDtypeStruct((N, D), jnp.float32),
        scratch_shapes=[
            pltpu.VMEM((N, D), jnp.float32),
            pltpu.VMEM((2, D), jnp.float32),
        ],
    )(h, p, w1, b1, w2, b2, g, b, batch2d)


def kernel(x, edge_index, batch,
           l0_w1, l0_b1, l0_w2, l0_b2,
           l1_w1, l1_b1, l1_w2, l1_b2,
           l2_w1, l2_b1, l2_w2, l2_b2,
           bn0_g, bn0_b, bn1_g, bn1_b, bn2_g, bn2_b):
    pad = E_PAD - E
    pad_src = (jnp.arange(pad, dtype=jnp.int32) * 13) % N
    pad_dst = N + jnp.arange(pad, dtype=jnp.int32) % (NP - N)
    se = jnp.stack([
        jnp.concatenate([edge_index[0], pad_src]),
        jnp.concatenate([edge_index[1], pad_dst]),
    ]).reshape(2, E_PAD // K, K)

    def row(a):
        return a.reshape(1, -1)

    p = _segsum(x, se)
    h = _mlp_bn(x, p, l0_w1, row(l0_b1), l0_w2, row(l0_b2), row(bn0_g), row(bn0_b))
    p = _segsum(h, se)
    h = _mlp_bn(h, p, l1_w1, row(l1_b1), l1_w2, row(l1_b2), row(bn1_g), row(bn1_b))
    p = _segsum(h, se)
    w2p = jnp.pad(l2_w2, ((0, 0), (0, D - 1)))
    b2p = jnp.pad(l2_b2.reshape(1, 1), ((0, 0), (0, D - 1)))
    g2 = jnp.broadcast_to(bn2_g.reshape(1, 1), (1, D))
    b2 = jnp.broadcast_to(bn2_b.reshape(1, 1), (1, D))
    out = _mlp_soft(h, p, l2_w1, row(l2_b1), w2p, b2p, g2, b2, batch.reshape(N, 1))
    return out[:, :1]


# speedup vs baseline: 1.3041x; 1.0257x over previous
import functools

import numpy as np

import jax
import jax.numpy as jnp
from jax import lax
from jax.experimental import pallas as pl
from jax.experimental.pallas import tpu as pltpu
from jax.experimental.pallas import tpu_sc as plsc

N = 10000
E = 320000
D = 128
G = 64

NC = 2
NS = 16
NW = NC * NS
EPW = E // NW
K = 128
CPW = 80
CPH = CPW // 2
E_PAD = NW * CPW * K
RPT = 624
NP = 15 * RPT + 664


def _segsum_body(h_hbm, se_hbm, out_hbm, idx, rowsb, acc, g0, g1):
    c = lax.axis_index("c")
    s = lax.axis_index("s")
    wid = s * NC + c

    rows0 = rowsb.at[0]
    rows1 = rowsb.at[1]

    zv = jnp.zeros((16,), jnp.float32)

    def zrow(r, carry):
        for j in range(D // 16):
            rowsb[0, r, pl.ds(j * 16, 16)] = zv
        return carry

    lax.fori_loop(0, K, zrow, 0)

    base = s * RPT

    @pl.when(s < NS - 1)
    def _():
        for off, size in ((0, 128), (128, 128), (256, 128), (384, 128), (512, 112)):
            pltpu.sync_copy(rowsb.at[0, pl.ds(0, size)], acc.at[pl.ds(base + off, size)])

    @pl.when(s == NS - 1)
    def _():
        for off, size in ((0, 128), (128, 128), (256, 128), (384, 128), (512, 128), (640, 24)):
            pltpu.sync_copy(rowsb.at[0, pl.ds(0, size)], acc.at[pl.ds(base + off, size)])

    for half in range(2):
        pltpu.sync_copy(se_hbm.at[:, pl.ds(wid * CPW + half * CPH, CPH)], idx)
        pltpu.async_copy(h_hbm.at[idx.at[0, 0]], rows0, g0)
        if half == 0:
            plsc.subcore_barrier()

        def body(jj, carry):
            j0 = 2 * jj
            j1 = j0 + 1
            pltpu.async_copy(h_hbm.at[idx.at[0, j1]], rows1, g1)
            pltpu.make_async_copy(h_hbm.at[idx.at[0, j0]], rows0, g0).wait()
            pltpu.sync_copy(rows0, acc.at[idx.at[1, j0]], add=True)

            @pl.when(j0 + 2 < CPH)
            def _():
                pltpu.async_copy(h_hbm.at[idx.at[0, j0 + 2]], rows0, g0)

            pltpu.make_async_copy(h_hbm.at[idx.at[0, j1]], rows1, g1).wait()
            pltpu.sync_copy(rows1, acc.at[idx.at[1, j1]], add=True)
            return carry

        lax.fori_loop(0, CPH // 2, body, 0)

    plsc.subcore_barrier()

    @pl.when(s < NS - 1)
    def _():
        pltpu.sync_copy(acc.at[pl.ds(base, RPT)], out_hbm.at[c, pl.ds(base, RPT)])

    @pl.when(s == NS - 1)
    def _():
        pltpu.sync_copy(acc.at[pl.ds(base, 664)], out_hbm.at[c, pl.ds(base, 664)])


@functools.cache
def _make_segsum():
    return pl.kernel(
        _segsum_body,
        out_type=jax.ShapeDtypeStruct((NC, NP, D), jnp.float32),
        mesh=plsc.VectorSubcoreMesh(
            core_axis_name="c", subcore_axis_name="s", num_cores=NC, num_subcores=NS
        ),
        scratch_types=[
            pltpu.VMEM((2, CPH, K), jnp.int32),
            pltpu.VMEM((2, K, D), jnp.float32),
            pltpu.VMEM_SHARED((NP, D), jnp.float32),
            pltpu.SemaphoreType.DMA,
            pltpu.SemaphoreType.DMA,
        ],
    )


def _segsum(h, se):
    return _make_segsum()(h, se)


R = 2000
NB = N // R


def _mlp_bn_body(h_ref, p_ref, w1_ref, b1_ref, w2_ref, b2_ref, g_ref, b_ref,
                 o_ref, m_s, st_s):
    ph = pl.program_id(0)
    i = pl.program_id(1)

    @pl.when(ph == 0)
    def _():
        z = h_ref[...] + p_ref[0] + p_ref[1]
        a = jnp.maximum(
            jnp.dot(z, w1_ref[...], preferred_element_type=jnp.float32) + b1_ref[...],
            0.0,
        )
        m = jnp.dot(a, w2_ref[...], preferred_element_type=jnp.float32) + b2_ref[...]
        m_s[pl.ds(i * R, R), :] = m
        st = jnp.concatenate(
            [jnp.sum(m, axis=0, keepdims=True), jnp.sum(m * m, axis=0, keepdims=True)],
            axis=0,
        )

        @pl.when(i == 0)
        def _():
            st_s[...] = jnp.zeros_like(st_s)

        st_s[...] += st
        o_ref[...] = m

    @pl.when(ph == 1)
    def _():
        mu = st_s[0:1, :] * (1.0 / N)
        ex2 = st_s[1:2, :] * (1.0 / N)
        var = ex2 - mu * mu
        m = m_s[pl.ds(i * R, R), :]
        v = (m - mu) * lax.rsqrt(var + 1e-5) * g_ref[...] + b_ref[...]
        o_ref[...] = jnp.maximum(v, 0.0)


def _mlp_bn(h, p, w1, b1, w2, b2, g, b):
    return pl.pallas_call(
        _mlp_bn_body,
        grid=(2, NB),
        in_specs=[
            pl.BlockSpec((R, D), lambda ph, i: (i, 0)),
            pl.BlockSpec((NC, R, D), lambda ph, i: (0, i, 0)),
            pl.BlockSpec((D, D), lambda ph, i: (0, 0)),
            pl.BlockSpec((1, D), lambda ph, i: (0, 0)),
            pl.BlockSpec((D, D), lambda ph, i: (0, 0)),
            pl.BlockSpec((1, D), lambda ph, i: (0, 0)),
            pl.BlockSpec((1, D), lambda ph, i: (0, 0)),
            pl.BlockSpec((1, D), lambda ph, i: (0, 0)),
        ],
        out_specs=pl.BlockSpec((R, D), lambda ph, i: (i, 0)),
        out_shape=jax.ShapeDtypeStruct((N, D), jnp.float32),
        scratch_shapes=[
            pltpu.VMEM((N, D), jnp.float32),
            pltpu.VMEM((2, D), jnp.float32),
        ],
    )(h, p, w1, b1, w2, b2, g, b)


def _mlp_soft_body(h_ref, p_ref, w1_ref, b1_ref, w2_ref, b2_ref, g_ref, b_ref,
                   batch_ref, o_ref, m_s, st_s):
    ph = pl.program_id(0)
    i = pl.program_id(1)

    @pl.when(ph == 0)
    def _():
        z = h_ref[...] + p_ref[0] + p_ref[1]
        a = jnp.maximum(
            jnp.dot(z, w1_ref[...], preferred_element_type=jnp.float32) + b1_ref[...],
            0.0,
        )
        m = jnp.dot(a, w2_ref[...], preferred_element_type=jnp.float32) + b2_ref[...]
        m_s[pl.ds(i * R, R), :] = m
        st = jnp.concatenate(
            [jnp.sum(m, axis=0, keepdims=True), jnp.sum(m * m, axis=0, keepdims=True)],
            axis=0,
        )

        @pl.when(i == 0)
        def _():
            st_s[...] = jnp.zeros_like(st_s)

        st_s[...] += st

    @pl.when((ph == 1) & (i == 0))
    def _():
        mu = st_s[0:1, 0:1] * (1.0 / N)
        ex2 = st_s[1:2, 0:1] * (1.0 / N)
        var = ex2 - mu * mu
        v = (m_s[...] - mu) * lax.rsqrt(var + 1e-5) * g_ref[...] + b_ref[...]
        s1 = v * (1.0 / 5.0)
        s0 = jnp.broadcast_to(s1, (N, D))
        gid = lax.broadcasted_iota(jnp.int32, (N, D), 1)
        onehot = batch_ref[...] == gid
        neg = jnp.full((N, D), -jnp.inf, jnp.float32)
        mx_g = jnp.max(jnp.where(onehot, s0, neg), axis=0, keepdims=True)
        mxb = jnp.sum(
            jnp.where(onehot, jnp.broadcast_to(mx_g, (N, D)), 0.0),
            axis=1, keepdims=True,
        )
        e1 = jnp.exp(s1 - mxb)
        den_g = jnp.sum(
            jnp.where(onehot, jnp.broadcast_to(e1, (N, D)), 0.0), axis=0, keepdims=True
        )
        denb = jnp.sum(
            jnp.where(onehot, jnp.broadcast_to(den_g, (N, D)), 0.0),
            axis=1, keepdims=True,
        )
        o_ref[...] = e1 / (denb + 1e-16)


def _mlp_soft(h, p, w1, b1, w2, b2, g, b, batch2d):
    return pl.pallas_call(
        _mlp_soft_body,
        grid=(2, NB),
        in_specs=[
            pl.BlockSpec((R, D), lambda ph, i: (i, 0)),
            pl.BlockSpec((NC, R, D), lambda ph, i: (0, i, 0)),
            pl.BlockSpec((D, D), lambda ph, i: (0, 0)),
            pl.BlockSpec((1, D), lambda ph, i: (0, 0)),
            pl.BlockSpec((D, 1), lambda ph, i: (0, 0)),
            pl.BlockSpec((1, 1), lambda ph, i: (0, 0)),
            pl.BlockSpec((1, 1), lambda ph, i: (0, 0)),
            pl.BlockSpec((1, 1), lambda ph, i: (0, 0)),
            pl.BlockSpec((N, 1), lambda ph, i: (0, 0)),
        ],
        out_specs=pl.BlockSpec((N, 1), lambda ph, i: (0, 0)),
        out_shape=jax.ShapeDtypeStruct((N, 1), jnp.float32),
        scratch_shapes=[
            pltpu.VMEM((N, 1), jnp.float32),
            pltpu.VMEM((2, 1), jnp.float32),
        ],
    )(h, p, w1, b1, w2, b2, g, b, batch2d)


_PAD_EDGES = np.stack([
    (np.arange(E_PAD - E, dtype=np.int32) * 13) % N,
    N + np.arange(E_PAD - E, dtype=np.int32) % (NP - N),
])


def kernel(x, edge_index, batch,
           l0_w1, l0_b1, l0_w2, l0_b2,
           l1_w1, l1_b1, l1_w2, l1_b2,
           l2_w1, l2_b1, l2_w2, l2_b2,
           bn0_g, bn0_b, bn1_g, bn1_b, bn2_g, bn2_b):
    se = jnp.concatenate([edge_index, _PAD_EDGES], axis=1).reshape(2, E_PAD // K, K)

    def row(a):
        return a.reshape(1, -1)

    p = _segsum(x, se)
    h = _mlp_bn(x, p, l0_w1, row(l0_b1), l0_w2, row(l0_b2), row(bn0_g), row(bn0_b))
    p = _segsum(h, se)
    h = _mlp_bn(h, p, l1_w1, row(l1_b1), l1_w2, row(l1_b2), row(bn1_g), row(bn1_b))
    p = _segsum(h, se)
    return _mlp_soft(h, p, l2_w1, row(l2_b1), l2_w2, l2_b2.reshape(1, 1),
                     bn2_g.reshape(1, 1), bn2_b.reshape(1, 1), batch.reshape(N, 1))
